# Initial kernel scaffold; baseline (speedup 1.0000x reference)
import jax
import jax.numpy as jnp
from jax.experimental import pallas as pl


def kernel(features, neigh_idx, W_pool0, b_pool0, W_fc0, b_fc0, bn_gamma, bn_beta, W_pool1, b_pool1, W_fc1, b_fc1):
    raise NotImplementedError("write your pallas kernel here")


# speedup vs baseline: 1.7941x; 1.7941x over previous
import functools

import jax
import jax.numpy as jnp
from jax import lax
from jax.experimental import pallas as pl
from jax.experimental.pallas import tpu as pltpu
from jax.experimental.pallas import tpu_sc as plsc

N = 50000
D = 256
S = 5

BM = 2000
GRID = N // BM

NC = 2
NS = 16
NW = NC * NS
B = 64
NPAD = 51200
CHUNK = NPAD // NW
T = CHUNK // B


def _gather_max_body(r_hbm, idx_hbm, out_hbm, idx_v, rows_v, out_v, sem):
    cid = lax.axis_index("c")
    sid = lax.axis_index("s")
    wid = sid * NC + cid

    def batch_body(t, carry):
        base = wid * CHUNK + t * B
        pltpu.sync_copy(idx_hbm.at[pl.ds(base * S, S * B)], idx_v)
        copies = [
            pltpu.async_copy(r_hbm.at[idx_v.at[pl.ds(s * B, B)]],
                             rows_v.at[s], sem)
            for s in range(S)
        ]
        for c in copies:
            c.wait()

        def node_body(i, carry2):
            for c in range(D // 16):
                sl = pl.ds(c * 16, 16)
                v = rows_v[0, i, sl]
                for s in range(1, S):
                    v = jnp.maximum(v, rows_v[s, i, sl])
                out_v[i, sl] = v
            return carry2

        lax.fori_loop(0, B, node_body, 0)
        pltpu.sync_copy(out_v, out_hbm.at[pl.ds(base, B)])
        return carry

    lax.fori_loop(0, T, batch_body, 0)


@functools.cache
def _gather_max():
    return pl.kernel(
        _gather_max_body,
        out_type=jax.ShapeDtypeStruct((NPAD, D), jnp.float32),
        mesh=plsc.VectorSubcoreMesh(core_axis_name="c", subcore_axis_name="s",
                                    num_cores=NC, num_subcores=NS),
        scratch_types=[
            pltpu.VMEM((S * B,), jnp.int32),
            pltpu.VMEM((S, B, D), jnp.float32),
            pltpu.VMEM((B, D), jnp.float32),
            pltpu.SemaphoreType.DMA,
        ],
    )


def _gather_max_call(r, idx_lin):
    return _gather_max()(r, idx_lin)


def _stage_a_body(x_ref, wp_ref, bp_ref, wf_ref, bf_ref, r0_ref, h1_ref):
    x = x_ref[...]
    zp = jnp.dot(x, wp_ref[...], preferred_element_type=jnp.float32)
    r0_ref[...] = jnp.maximum(zp + bp_ref[...], 0.0)
    zf = jnp.dot(x, wf_ref[...], preferred_element_type=jnp.float32)
    h1_ref[...] = zf + bf_ref[...]


def _stage_c_body(h1_ref, a_ref, w_ref, h_ref, sum_ref, sq_ref):
    i = pl.program_id(0)
    za = jnp.dot(a_ref[...], w_ref[...], preferred_element_type=jnp.float32)
    h = jnp.maximum(h1_ref[...] + za, 0.0)
    h_ref[...] = h
    ps = jnp.sum(h.reshape(BM // 8, 8, D), axis=0)
    pq = jnp.sum((h * h).reshape(BM // 8, 8, D), axis=0)

    @pl.when(i == 0)
    def _():
        sum_ref[...] = ps
        sq_ref[...] = pq

    @pl.when(i > 0)
    def _():
        sum_ref[...] += ps
        sq_ref[...] += pq


def _stage_d_body(h_ref, sum_ref, sq_ref, g_ref, bt_ref, wp_ref, bp_ref,
                  wf_ref, bf_ref, out1_ref, r1_ref, h2_ref):
    s = jnp.sum(sum_ref[...], axis=0, keepdims=True)
    sq = jnp.sum(sq_ref[...], axis=0, keepdims=True)
    mean = s * (1.0 / N)
    var = sq * (1.0 / N) - mean * mean
    inv = lax.rsqrt(var + 1e-5)
    y = (h_ref[...] - mean) * (inv * g_ref[...]) + bt_ref[...]
    nrm = jnp.sqrt(jnp.sum(y * y, axis=1, keepdims=True))
    z = y / (nrm + 1e-6)
    out1_ref[...] = z
    zp = jnp.dot(z, wp_ref[...], preferred_element_type=jnp.float32)
    r1_ref[...] = jnp.maximum(zp + bp_ref[...], 0.0)
    zf = jnp.dot(z, wf_ref[...], preferred_element_type=jnp.float32)
    h2_ref[...] = zf + bf_ref[...]


def _stage_f_body(h2_ref, a_ref, w_ref, out_ref):
    za = jnp.dot(a_ref[...], w_ref[...], preferred_element_type=jnp.float32)
    out_ref[...] = h2_ref[...] + za


_row_spec = pl.BlockSpec((BM, D), lambda i: (i, 0))
_w_spec = pl.BlockSpec((D, D), lambda i: (0, 0))
_vec_spec = pl.BlockSpec((1, D), lambda i: (0, 0))
_acc_spec = pl.BlockSpec((8, D), lambda i: (0, 0))

_f32 = jnp.float32


def _stage_a(x, wp, bp, wf, bf):
    return pl.pallas_call(
        _stage_a_body,
        grid=(GRID,),
        in_specs=[_row_spec, _w_spec, _vec_spec, _w_spec, _vec_spec],
        out_specs=[_row_spec, _row_spec],
        out_shape=[jax.ShapeDtypeStruct((N, D), _f32)] * 2,
    )(x, wp, bp, wf, bf)


def _stage_c(h1, agg0, w):
    return pl.pallas_call(
        _stage_c_body,
        grid=(GRID,),
        in_specs=[_row_spec, _row_spec, _w_spec],
        out_specs=[_row_spec, _acc_spec, _acc_spec],
        out_shape=[
            jax.ShapeDtypeStruct((N, D), _f32),
            jax.ShapeDtypeStruct((8, D), _f32),
            jax.ShapeDtypeStruct((8, D), _f32),
        ],
    )(h1, agg0, w)


def _stage_d(h, sums, sq, g, bt, wp, bp, wf, bf):
    return pl.pallas_call(
        _stage_d_body,
        grid=(GRID,),
        in_specs=[_row_spec, _acc_spec, _acc_spec, _vec_spec, _vec_spec,
                  _w_spec, _vec_spec, _w_spec, _vec_spec],
        out_specs=[_row_spec, _row_spec, _row_spec],
        out_shape=[jax.ShapeDtypeStruct((N, D), _f32)] * 3,
    )(h, sums, sq, g, bt, wp, bp, wf, bf)


def _stage_f(h2, agg1, w):
    return pl.pallas_call(
        _stage_f_body,
        grid=(GRID,),
        in_specs=[_row_spec, _row_spec, _w_spec],
        out_specs=_row_spec,
        out_shape=jax.ShapeDtypeStruct((N, D), _f32),
    )(h2, agg1, w)


def kernel(features, neigh_idx, W_pool0, b_pool0, W_fc0, b_fc0, bn_gamma,
           bn_beta, W_pool1, b_pool1, W_fc1, b_fc1):
    idx = neigh_idx.astype(jnp.int32)
    idx_pad = jnp.pad(idx, ((0, NPAD - N), (0, 0)))
    idx_lin = idx_pad.reshape(NW, T, B, S).transpose(0, 1, 3, 2).reshape(-1)

    bp0 = b_pool0.reshape(1, D)
    bp1 = b_pool1.reshape(1, D)
    bf0 = b_fc0.reshape(1, D)
    bf1 = b_fc1.reshape(1, D)
    g = bn_gamma.reshape(1, D)
    bt = bn_beta.reshape(1, D)
    wf0a, wf0b = W_fc0[:D], W_fc0[D:]
    wf1a, wf1b = W_fc1[:D], W_fc1[D:]

    r0, h1 = _stage_a(features, W_pool0, bp0, wf0a, bf0)
    agg0 = _gather_max_call(r0, idx_lin)[:N]
    h, sums, sq = _stage_c(h1, agg0, wf0b)
    out1, r1, h2 = _stage_d(h, sums, sq, g, bt, W_pool1, bp1, wf1a, bf1)
    agg1 = _gather_max_call(r1, idx_lin)[:N]
    return _stage_f(h2, agg1, wf1b)


# speedup vs baseline: 2.0503x; 1.1428x over previous
import functools

import jax
import jax.numpy as jnp
from jax import lax
from jax.experimental import pallas as pl
from jax.experimental.pallas import tpu as pltpu
from jax.experimental.pallas import tpu_sc as plsc

N = 50000
D = 256
S = 5

BM = 2000
GRID = N // BM

NC = 2
NS = 16
NW = NC * NS
B = 32
NPAD = 51200
CHUNK = NPAD // NW
T = CHUNK // B


def _gather_max_body(r_hbm, idx_hbm, out_hbm, idx_v0, idx_v1, rows_v0,
                     rows_v1, out_v0, out_v1, gsem0, gsem1, osem0, osem1):
    cid = lax.axis_index("c")
    sid = lax.axis_index("s")
    wid = sid * NC + cid
    idx_v = (idx_v0, idx_v1)
    rows_v = (rows_v0, rows_v1)
    out_v = (out_v0, out_v1)
    gsem = (gsem0, gsem1)
    osem = (osem0, osem1)

    def issue(buf, t):
        base = wid * CHUNK + t * B
        pltpu.sync_copy(idx_hbm.at[pl.ds(base * S, S * B)], idx_v[buf])
        for s in range(S):
            pltpu.async_copy(
                r_hbm.at[idx_v[buf].at[pl.ds(s * B, B)]],
                rows_v[buf].at[s], gsem[buf])

    def consume(buf, t):
        base = wid * CHUNK + t * B
        for s in range(S):
            pltpu.make_async_copy(
                r_hbm.at[idx_v[buf].at[pl.ds(s * B, B)]],
                rows_v[buf].at[s], gsem[buf]).wait()

        @pl.when(t >= 2)
        def _():
            pltpu.make_async_copy(
                out_v[buf], out_hbm.at[pl.ds(base, B)], osem[buf]).wait()

        def node_body(i, carry2):
            for c in range(D // 16):
                sl = pl.ds(c * 16, 16)
                v = rows_v[buf][0, i, sl]
                for s in range(1, S):
                    v = jnp.maximum(v, rows_v[buf][s, i, sl])
                out_v[buf][i, sl] = v
            return carry2

        lax.fori_loop(0, B, node_body, 0)
        pltpu.async_copy(out_v[buf], out_hbm.at[pl.ds(base, B)], osem[buf])

    issue(0, 0)
    issue(1, 1)

    def pair_body(g, carry):
        t0 = g * 2
        t1 = t0 + 1
        consume(0, t0)

        @pl.when(t0 + 2 < T)
        def _():
            issue(0, t0 + 2)

        consume(1, t1)

        @pl.when(t1 + 2 < T)
        def _():
            issue(1, t1 + 2)

        return carry

    lax.fori_loop(0, T // 2, pair_body, 0)
    for buf in range(2):
        base = wid * CHUNK + (T - 2 + buf) * B
        pltpu.make_async_copy(
            out_v[buf], out_hbm.at[pl.ds(base, B)], osem[buf]).wait()


@functools.cache
def _gather_max():
    return pl.kernel(
        _gather_max_body,
        out_type=jax.ShapeDtypeStruct((NPAD, D), jnp.float32),
        mesh=plsc.VectorSubcoreMesh(core_axis_name="c", subcore_axis_name="s",
                                    num_cores=NC, num_subcores=NS),
        scratch_types=[
            pltpu.VMEM((S * B,), jnp.int32),
            pltpu.VMEM((S * B,), jnp.int32),
            pltpu.VMEM((S, B, D), jnp.float32),
            pltpu.VMEM((S, B, D), jnp.float32),
            pltpu.VMEM((B, D), jnp.float32),
            pltpu.VMEM((B, D), jnp.float32),
            pltpu.SemaphoreType.DMA,
            pltpu.SemaphoreType.DMA,
            pltpu.SemaphoreType.DMA,
            pltpu.SemaphoreType.DMA,
        ],
    )


def _gather_max_call(r, idx_lin):
    return _gather_max()(r, idx_lin)


def _stage_a_body(x_ref, wp_ref, bp_ref, wf_ref, bf_ref, r0_ref, h1_ref):
    x = x_ref[...]
    zp = jnp.dot(x, wp_ref[...], preferred_element_type=jnp.float32)
    r0_ref[...] = jnp.maximum(zp + bp_ref[...], 0.0)
    zf = jnp.dot(x, wf_ref[...], preferred_element_type=jnp.float32)
    h1_ref[...] = zf + bf_ref[...]


def _stage_c_body(h1_ref, a_ref, w_ref, h_ref, sum_ref, sq_ref):
    i = pl.program_id(0)
    za = jnp.dot(a_ref[...], w_ref[...], preferred_element_type=jnp.float32)
    h = jnp.maximum(h1_ref[...] + za, 0.0)
    h_ref[...] = h
    ps = jnp.sum(h.reshape(BM // 8, 8, D), axis=0)
    pq = jnp.sum((h * h).reshape(BM // 8, 8, D), axis=0)

    @pl.when(i == 0)
    def _():
        sum_ref[...] = ps
        sq_ref[...] = pq

    @pl.when(i > 0)
    def _():
        sum_ref[...] += ps
        sq_ref[...] += pq


def _stage_d_body(h_ref, sum_ref, sq_ref, g_ref, bt_ref, wp_ref, bp_ref,
                  wf_ref, bf_ref, out1_ref, r1_ref, h2_ref):
    s = jnp.sum(sum_ref[...], axis=0, keepdims=True)
    sq = jnp.sum(sq_ref[...], axis=0, keepdims=True)
    mean = s * (1.0 / N)
    var = sq * (1.0 / N) - mean * mean
    inv = lax.rsqrt(var + 1e-5)
    y = (h_ref[...] - mean) * (inv * g_ref[...]) + bt_ref[...]
    nrm = jnp.sqrt(jnp.sum(y * y, axis=1, keepdims=True))
    z = y / (nrm + 1e-6)
    out1_ref[...] = z
    zp = jnp.dot(z, wp_ref[...], preferred_element_type=jnp.float32)
    r1_ref[...] = jnp.maximum(zp + bp_ref[...], 0.0)
    zf = jnp.dot(z, wf_ref[...], preferred_element_type=jnp.float32)
    h2_ref[...] = zf + bf_ref[...]


def _stage_f_body(h2_ref, a_ref, w_ref, out_ref):
    za = jnp.dot(a_ref[...], w_ref[...], preferred_element_type=jnp.float32)
    out_ref[...] = h2_ref[...] + za


_row_spec = pl.BlockSpec((BM, D), lambda i: (i, 0))
_w_spec = pl.BlockSpec((D, D), lambda i: (0, 0))
_vec_spec = pl.BlockSpec((1, D), lambda i: (0, 0))
_acc_spec = pl.BlockSpec((8, D), lambda i: (0, 0))

_f32 = jnp.float32


def _stage_a(x, wp, bp, wf, bf):
    return pl.pallas_call(
        _stage_a_body,
        grid=(GRID,),
        in_specs=[_row_spec, _w_spec, _vec_spec, _w_spec, _vec_spec],
        out_specs=[_row_spec, _row_spec],
        out_shape=[jax.ShapeDtypeStruct((N, D), _f32)] * 2,
    )(x, wp, bp, wf, bf)


def _stage_c(h1, agg0, w):
    return pl.pallas_call(
        _stage_c_body,
        grid=(GRID,),
        in_specs=[_row_spec, _row_spec, _w_spec],
        out_specs=[_row_spec, _acc_spec, _acc_spec],
        out_shape=[
            jax.ShapeDtypeStruct((N, D), _f32),
            jax.ShapeDtypeStruct((8, D), _f32),
            jax.ShapeDtypeStruct((8, D), _f32),
        ],
    )(h1, agg0, w)


def _stage_d(h, sums, sq, g, bt, wp, bp, wf, bf):
    return pl.pallas_call(
        _stage_d_body,
        grid=(GRID,),
        in_specs=[_row_spec, _acc_spec, _acc_spec, _vec_spec, _vec_spec,
                  _w_spec, _vec_spec, _w_spec, _vec_spec],
        out_specs=[_row_spec, _row_spec, _row_spec],
        out_shape=[jax.ShapeDtypeStruct((N, D), _f32)] * 3,
    )(h, sums, sq, g, bt, wp, bp, wf, bf)


def _stage_f(h2, agg1, w):
    return pl.pallas_call(
        _stage_f_body,
        grid=(GRID,),
        in_specs=[_row_spec, _row_spec, _w_spec],
        out_specs=_row_spec,
        out_shape=jax.ShapeDtypeStruct((N, D), _f32),
    )(h2, agg1, w)


def kernel(features, neigh_idx, W_pool0, b_pool0, W_fc0, b_fc0, bn_gamma,
           bn_beta, W_pool1, b_pool1, W_fc1, b_fc1):
    idx = neigh_idx.astype(jnp.int32)
    idx_pad = jnp.pad(idx, ((0, NPAD - N), (0, 0)))
    idx_lin = idx_pad.reshape(NW, T, B, S).transpose(0, 1, 3, 2).reshape(-1)

    bp0 = b_pool0.reshape(1, D)
    bp1 = b_pool1.reshape(1, D)
    bf0 = b_fc0.reshape(1, D)
    bf1 = b_fc1.reshape(1, D)
    g = bn_gamma.reshape(1, D)
    bt = bn_beta.reshape(1, D)
    wf0a, wf0b = W_fc0[:D], W_fc0[D:]
    wf1a, wf1b = W_fc1[:D], W_fc1[D:]

    r0, h1 = _stage_a(features, W_pool0, bp0, wf0a, bf0)
    agg0 = _gather_max_call(r0, idx_lin)
    h, sums, sq = _stage_c(h1, agg0, wf0b)
    out1, r1, h2 = _stage_d(h, sums, sq, g, bt, W_pool1, bp1, wf1a, bf1)
    agg1 = _gather_max_call(r1, idx_lin)
    return _stage_f(h2, agg1, wf1b)


# speedup vs baseline: 3.7168x; 1.8128x over previous
import functools

import jax
import jax.numpy as jnp
from jax import lax
from jax.experimental import pallas as pl
from jax.experimental.pallas import tpu as pltpu
from jax.experimental.pallas import tpu_sc as plsc

N = 50000
D = 256
S = 5

BM = 2000
GRID = N // BM

NC = 2
NS = 16
NW = NC * NS
B = 32
T0 = 76
T1 = 22
TOTAL_BATCHES = NS * (T0 + T1)
NPAD = TOTAL_BATCHES * B


def _gather_max_body(r_hbm, idx_hbm, out_hbm, idx_v0, idx_v1, rows_v0,
                     rows_v1, out_v0, out_v1, gsem0, gsem1, osem0, osem1):
    cid = lax.axis_index("c")
    sid = lax.axis_index("s")
    tc = jnp.where(cid == 0, T0, T1)
    wbb = cid * (NS * T0) + sid * tc
    idx_v = (idx_v0, idx_v1)
    rows_v = (rows_v0, rows_v1)
    out_v = (out_v0, out_v1)
    gsem = (gsem0, gsem1)
    osem = (osem0, osem1)

    def issue(buf, t):
        base = (wbb + t) * B
        pltpu.sync_copy(idx_hbm.at[pl.ds(base * S, S * B)], idx_v[buf])
        for s in range(S):
            pltpu.async_copy(
                r_hbm.at[idx_v[buf].at[pl.ds(s * B, B)]],
                rows_v[buf].at[s], gsem[buf])

    def consume(buf, t):
        base = (wbb + t) * B
        for s in range(S):
            pltpu.make_async_copy(
                r_hbm.at[idx_v[buf].at[pl.ds(s * B, B)]],
                rows_v[buf].at[s], gsem[buf]).wait()

        @pl.when(t >= 2)
        def _():
            pltpu.make_async_copy(
                out_v[buf], out_hbm.at[pl.ds(base, B)], osem[buf]).wait()

        def node_body(i, carry2):
            for c in range(D // 16):
                sl = pl.ds(c * 16, 16)
                v = rows_v[buf][0, i, sl]
                for s in range(1, S):
                    v = jnp.maximum(v, rows_v[buf][s, i, sl])
                out_v[buf][i, sl] = v
            return carry2

        lax.fori_loop(0, B, node_body, 0)
        pltpu.async_copy(out_v[buf], out_hbm.at[pl.ds(base, B)], osem[buf])

    issue(0, 0)
    issue(1, 1)

    def pair_body(g, carry):
        t0 = g * 2
        t1 = t0 + 1
        consume(0, t0)

        @pl.when(t0 + 2 < tc)
        def _():
            issue(0, t0 + 2)

        consume(1, t1)

        @pl.when(t1 + 2 < tc)
        def _():
            issue(1, t1 + 2)

        return carry

    lax.fori_loop(0, lax.div(tc, 2), pair_body, 0)
    for buf in range(2):
        base = (wbb + tc - 2 + buf) * B
        pltpu.make_async_copy(
            out_v[buf], out_hbm.at[pl.ds(base, B)], osem[buf]).wait()


@functools.cache
def _gather_max():
    return pl.kernel(
        _gather_max_body,
        out_type=jax.ShapeDtypeStruct((NPAD, D), jnp.float32),
        mesh=plsc.VectorSubcoreMesh(core_axis_name="c", subcore_axis_name="s",
                                    num_cores=NC, num_subcores=NS),
        scratch_types=[
            pltpu.VMEM((S * B,), jnp.int32),
            pltpu.VMEM((S * B,), jnp.int32),
            pltpu.VMEM((S, B, D), jnp.float32),
            pltpu.VMEM((S, B, D), jnp.float32),
            pltpu.VMEM((B, D), jnp.float32),
            pltpu.VMEM((B, D), jnp.float32),
            pltpu.SemaphoreType.DMA,
            pltpu.SemaphoreType.DMA,
            pltpu.SemaphoreType.DMA,
            pltpu.SemaphoreType.DMA,
        ],
    )


def _gather_max_call(r, idx_lin):
    return _gather_max()(r, idx_lin)


def _stage_a_body(x_ref, wp_ref, bp_ref, wf_ref, bf_ref, r0_ref, h1_ref):
    x = x_ref[...]
    zp = jnp.dot(x, wp_ref[...], preferred_element_type=jnp.float32)
    r0_ref[...] = jnp.maximum(zp + bp_ref[...], 0.0)
    zf = jnp.dot(x, wf_ref[...], preferred_element_type=jnp.float32)
    h1_ref[...] = zf + bf_ref[...]


def _stage_c_body(h1_ref, a_ref, w_ref, h_ref, sum_ref, sq_ref):
    i = pl.program_id(0)
    za = jnp.dot(a_ref[...], w_ref[...], preferred_element_type=jnp.float32)
    h = jnp.maximum(h1_ref[...] + za, 0.0)
    h_ref[...] = h
    ps = jnp.sum(h.reshape(BM // 8, 8, D), axis=0)
    pq = jnp.sum((h * h).reshape(BM // 8, 8, D), axis=0)

    @pl.when(i == 0)
    def _():
        sum_ref[...] = ps
        sq_ref[...] = pq

    @pl.when(i > 0)
    def _():
        sum_ref[...] += ps
        sq_ref[...] += pq


def _stage_d_body(h_ref, sum_ref, sq_ref, g_ref, bt_ref, wp_ref, bp_ref,
                  wf_ref, bf_ref, out1_ref, r1_ref, h2_ref):
    s = jnp.sum(sum_ref[...], axis=0, keepdims=True)
    sq = jnp.sum(sq_ref[...], axis=0, keepdims=True)
    mean = s * (1.0 / N)
    var = sq * (1.0 / N) - mean * mean
    inv = lax.rsqrt(var + 1e-5)
    y = (h_ref[...] - mean) * (inv * g_ref[...]) + bt_ref[...]
    nrm = jnp.sqrt(jnp.sum(y * y, axis=1, keepdims=True))
    z = y / (nrm + 1e-6)
    out1_ref[...] = z
    zp = jnp.dot(z, wp_ref[...], preferred_element_type=jnp.float32)
    r1_ref[...] = jnp.maximum(zp + bp_ref[...], 0.0)
    zf = jnp.dot(z, wf_ref[...], preferred_element_type=jnp.float32)
    h2_ref[...] = zf + bf_ref[...]


def _stage_f_body(h2_ref, a_ref, w_ref, out_ref):
    za = jnp.dot(a_ref[...], w_ref[...], preferred_element_type=jnp.float32)
    out_ref[...] = h2_ref[...] + za


_row_spec = pl.BlockSpec((BM, D), lambda i: (i, 0))
_w_spec = pl.BlockSpec((D, D), lambda i: (0, 0))
_vec_spec = pl.BlockSpec((1, D), lambda i: (0, 0))
_acc_spec = pl.BlockSpec((8, D), lambda i: (0, 0))

_f32 = jnp.float32


def _stage_a(x, wp, bp, wf, bf):
    return pl.pallas_call(
        _stage_a_body,
        grid=(GRID,),
        in_specs=[_row_spec, _w_spec, _vec_spec, _w_spec, _vec_spec],
        out_specs=[_row_spec, _row_spec],
        out_shape=[jax.ShapeDtypeStruct((N, D), _f32)] * 2,
    )(x, wp, bp, wf, bf)


def _stage_c(h1, agg0, w):
    return pl.pallas_call(
        _stage_c_body,
        grid=(GRID,),
        in_specs=[_row_spec, _row_spec, _w_spec],
        out_specs=[_row_spec, _acc_spec, _acc_spec],
        out_shape=[
            jax.ShapeDtypeStruct((N, D), _f32),
            jax.ShapeDtypeStruct((8, D), _f32),
            jax.ShapeDtypeStruct((8, D), _f32),
        ],
    )(h1, agg0, w)


def _stage_d(h, sums, sq, g, bt, wp, bp, wf, bf):
    return pl.pallas_call(
        _stage_d_body,
        grid=(GRID,),
        in_specs=[_row_spec, _acc_spec, _acc_spec, _vec_spec, _vec_spec,
                  _w_spec, _vec_spec, _w_spec, _vec_spec],
        out_specs=[_row_spec, _row_spec, _row_spec],
        out_shape=[jax.ShapeDtypeStruct((N, D), _f32)] * 3,
    )(h, sums, sq, g, bt, wp, bp, wf, bf)


def _stage_f(h2, agg1, w):
    return pl.pallas_call(
        _stage_f_body,
        grid=(GRID,),
        in_specs=[_row_spec, _row_spec, _w_spec],
        out_specs=_row_spec,
        out_shape=jax.ShapeDtypeStruct((N, D), _f32),
    )(h2, agg1, w)


def kernel(features, neigh_idx, W_pool0, b_pool0, W_fc0, b_fc0, bn_gamma,
           bn_beta, W_pool1, b_pool1, W_fc1, b_fc1):
    idx = neigh_idx.astype(jnp.int32)
    idx_pad = jnp.pad(idx, ((0, NPAD - N), (0, 0)))
    idx_lin = (idx_pad.reshape(TOTAL_BATCHES, B, S)
               .transpose(0, 2, 1).reshape(-1))

    bp0 = b_pool0.reshape(1, D)
    bp1 = b_pool1.reshape(1, D)
    bf0 = b_fc0.reshape(1, D)
    bf1 = b_fc1.reshape(1, D)
    g = bn_gamma.reshape(1, D)
    bt = bn_beta.reshape(1, D)
    wf0a, wf0b = W_fc0[:D], W_fc0[D:]
    wf1a, wf1b = W_fc1[:D], W_fc1[D:]

    r0, h1 = _stage_a(features, W_pool0, bp0, wf0a, bf0)
    agg0 = _gather_max_call(r0, idx_lin)
    h, sums, sq = _stage_c(h1, agg0, wf0b)
    out1, r1, h2 = _stage_d(h, sums, sq, g, bt, W_pool1, bp1, wf1a, bf1)
    agg1 = _gather_max_call(r1, idx_lin)
    return _stage_f(h2, agg1, wf1b)
